# Initial kernel scaffold; baseline (speedup 1.0000x reference)
import jax
import jax.numpy as jnp
from jax.experimental import pallas as pl


def kernel(x, aux, j1):
    raise NotImplementedError("write your pallas kernel here")


# speedup vs baseline: 4.5955x; 4.5955x over previous
import jax
import jax.numpy as jnp
from jax import lax
from jax.experimental import pallas as pl
from jax.experimental.pallas import tpu as pltpu
from jax.experimental.pallas import tpu_sc as plsc

_L = 8
_BATCH = 16384
_TABLE = 256
_NUM_WORKERS = 32
_BPW = _BATCH // _NUM_WORKERS
_LANES = 16
_STEPS = _BPW // _LANES


def _ewf_body(xT_hbm, aux_hbm, out_hbm, xbuf, table, outbuf):
    wid = lax.axis_index("s") * 2 + lax.axis_index("c")
    base = wid * _BPW
    pltpu.sync_copy(xT_hbm.at[:, pl.ds(base, _BPW)], xbuf)
    pltpu.sync_copy(aux_hbm, table)
    @pl.loop(0, _STEPS)
    def _step(s):
        sl = pl.ds(s * _LANES, _LANES)
        acc = jnp.zeros((_LANES,), jnp.int32)
        for i in range(_L):
            v = xbuf[i, sl]
            z = lax.rem(v + 1, 3) >> 1
            acc = acc * 2 + z
        outbuf[sl] = plsc.load_gather(table, [acc])
    pltpu.sync_copy(outbuf, out_hbm.at[pl.ds(base, _BPW)])


def kernel(x, aux, j1):
    xT = x.T
    mesh = plsc.VectorSubcoreMesh(core_axis_name="c", subcore_axis_name="s")
    return pl.kernel(
        _ewf_body,
        out_type=jax.ShapeDtypeStruct((_BATCH,), jnp.float32),
        mesh=mesh,
        compiler_params=pltpu.CompilerParams(needs_layout_passes=False),
        scratch_types=[
            pltpu.VMEM((_L, _BPW), jnp.int32),
            pltpu.VMEM((_TABLE,), jnp.float32),
            pltpu.VMEM((_BPW,), jnp.float32),
        ],
    )(xT, aux)


# speedup vs baseline: 6.3835x; 1.3891x over previous
import jax
import jax.numpy as jnp
from jax import lax
from jax.experimental import pallas as pl
from jax.experimental.pallas import tpu as pltpu
from jax.experimental.pallas import tpu_sc as plsc

_L = 8
_BATCH = 16384
_TABLE = 256
_NUM_WORKERS = 32
_BPW = _BATCH // _NUM_WORKERS
_LANES = 16
_STEPS = _BPW // _LANES


def _ewf_body(xT_hbm, aux_hbm, out_hbm, xbuf, table, outbuf):
    wid = lax.axis_index("s") * 2 + lax.axis_index("c")
    base = wid * _BPW
    pltpu.sync_copy(xT_hbm.at[:, pl.ds(base, _BPW)], xbuf)
    pltpu.sync_copy(aux_hbm, table)
    @pl.loop(0, _STEPS)
    def _step(s):
        sl = pl.ds(s * _LANES, _LANES)
        acc = jnp.zeros((_LANES,), jnp.int32)
        for i in range(_L):
            v = xbuf[i, sl]
            acc = (acc << 1) | (v & 1)
        outbuf[sl] = plsc.load_gather(table, [acc])
    pltpu.sync_copy(outbuf, out_hbm.at[pl.ds(base, _BPW)])


def kernel(x, aux, j1):
    xT = x.T
    mesh = plsc.VectorSubcoreMesh(core_axis_name="c", subcore_axis_name="s")
    return pl.kernel(
        _ewf_body,
        out_type=jax.ShapeDtypeStruct((_BATCH,), jnp.float32),
        mesh=mesh,
        compiler_params=pltpu.CompilerParams(needs_layout_passes=False),
        scratch_types=[
            pltpu.VMEM((_L, _BPW), jnp.int32),
            pltpu.VMEM((_TABLE,), jnp.float32),
            pltpu.VMEM((_BPW,), jnp.float32),
        ],
    )(xT, aux)
